# baseline (device time: 80067 ns/iter reference)
import jax
import jax.numpy as jnp
from jax import lax
from jax.experimental import pallas as pl
from jax.experimental.pallas import tpu as pltpu

N_DEV = 4
SQ = 1024
SQ_SH = SQ // N_DEV
H_LOC = 8
DH = 128
SKV = 4096
NPH = 4
KPP = SKV // NPH
D_MODEL = 1024
HD = D_MODEL // 2
SCALE = 0.08838834764831843

BF = jnp.bfloat16
F32 = jnp.float32


def kernel(x, Wq, K_ext, V_ext, Wo):
    def body(x_ref, wq_ref, k_ref, v_ref, wo_ref, out_ref,
             xfull, wq_bf, wo_bf, qh, ctxh, ctxg, pout,
             kst, vst, kre, vre, ones_b, sbuf, rbuf,
             dma_k, dma_v, agr_send, agr_recv, agl_send, agl_recv,
             rs_send, rs_recv):
        my = lax.axis_index("i")
        left = lax.rem(my + N_DEV - 1, N_DEV)
        right = lax.rem(my + 1, N_DEV)
        hb = my * H_LOC

        barrier = pltpu.get_barrier_semaphore()
        for nbr in (left, right):
            pl.semaphore_signal(barrier, inc=1, device_id=(nbr,),
                                device_id_type=pl.DeviceIdType.MESH)
        pl.semaphore_wait(barrier, 2)

        def kv_dma(h):
            s = h % 2
            dk = pltpu.make_async_copy(
                k_ref.at[0, :, pl.ds(hb + h, 1), :], kst.at[s], dma_k.at[s])
            dv = pltpu.make_async_copy(
                v_ref.at[0, :, pl.ds(hb + h, 1), :], vst.at[s], dma_v.at[s])
            return dk, dv

        def kv_start(h):
            dk, dv = kv_dma(h)
            dk.start()
            dv.start()

        def kv_finish(h):
            dk, dv = kv_dma(h)
            dk.wait()
            dv.wait()
            s = h % 2

            def reorg(a, _):
                for p in range(NPH):
                    src = a * 256 + p * 64
                    kre[h, p, pl.ds(a * 64, 64), :] = \
                        kst[s, pl.ds(src, 64), 0, :].astype(BF)
                    vre[h, p, pl.ds(a * 64, 64), :] = \
                        vst[s, pl.ds(src, 64), 0, :].astype(BF)
                return 0
            lax.fori_loop(0, SKV // 256, reorg, 0)

        def q_chunk(cidx, g, u):
            qc = (jnp.dot(
                xfull[pl.ds(cidx, 1)].reshape(SQ_SH, D_MODEL), wq_bf[...],
                preferred_element_type=F32,
            ) * SCALE).astype(BF)
            for h in range(H_LOC):
                qh[g, h, u] = qc[:, h * DH:(h + 1) * DH]

        def attn_group(g):
            for p in range(NPH):
                qb = qh[g, :, :, p * 64:(p + 1) * 64, :].reshape(
                    H_LOC, 2 * 64, DH)
                s = lax.dot_general(
                    qb, kre[:, p], (((2,), (2,)), ((0,), (0,))),
                    preferred_element_type=F32)
                e = jnp.exp(s).astype(BF)
                ctxb = lax.dot_general(
                    e, vre[:, p], (((2,), (1,)), ((0,), (0,))),
                    preferred_element_type=F32)
                denom = lax.dot_general(
                    e, ones_b[...], (((2,), (1,)), ((0,), (0,))),
                    preferred_element_type=F32)[:, :, 0:1]
                val = (ctxb / denom).astype(BF)
                for u in range(2):
                    row = u * SQ_SH + p * 64
                    ctxh[g, :, row:row + 64, :] = val[:, u * 64:(u + 1) * 64, :]
            for h in range(H_LOC):
                ctxg[g, :, h * DH:(h + 1) * DH] = ctxh[g, h]

        def rdma_ring(src, dst, send_sem, recv_sem, target):
            return pltpu.make_async_remote_copy(
                src_ref=src, dst_ref=dst, send_sem=send_sem,
                recv_sem=recv_sem, device_id=(target,),
                device_id_type=pl.DeviceIdType.MESH)

        ones_b[...] = jnp.ones((H_LOC, KPP, 8), BF)
        kv_start(0)
        kv_start(1)
        xfull[pl.ds(my, 1)] = x_ref[...].astype(BF)
        r0 = rdma_ring(xfull.at[pl.ds(my, 1)], xfull.at[pl.ds(my, 1)],
                       agr_send.at[0], agr_recv.at[0], right)
        l0 = rdma_ring(xfull.at[pl.ds(my, 1)], xfull.at[pl.ds(my, 1)],
                       agl_send.at[0], agl_recv.at[0], left)
        r0.start()
        l0.start()
        wq_bf[...] = wq_ref[...].astype(BF)
        wo_bf[...] = wo_ref[...].astype(BF)
        q_chunk(my, 1, 1)
        kv_finish(0)
        kv_start(2)
        r0.wait()
        cm1 = lax.rem(my - 1 + N_DEV, N_DEV)
        q_chunk(cm1, 0, 0)
        r1 = rdma_ring(xfull.at[pl.ds(cm1, 1)], xfull.at[pl.ds(cm1, 1)],
                       agr_send.at[1], agr_recv.at[1], right)
        r1.start()
        kv_finish(1)
        kv_start(3)
        kv_finish(2)
        kv_start(4)
        r1.wait()
        q_chunk(lax.rem(my - 2 + N_DEV, N_DEV), 0, 1)
        kv_finish(3)
        kv_start(5)
        kv_finish(4)
        kv_start(6)
        kv_finish(5)
        kv_start(7)
        kv_finish(6)
        kv_finish(7)

        attn_group(0)
        pout[0:2 * SQ_SH, :] = jnp.dot(ctxg[0], wo_bf[...],
                                       preferred_element_type=F32)

        l0.wait()
        q_chunk(lax.rem(my + 1, N_DEV), 1, 0)

        def rs_rdma(s, half):
            c0 = half * HD
            return rdma_ring(
                sbuf.at[:, pl.ds(c0, HD)], rbuf.at[s, :, pl.ds(c0, HD)],
                rs_send.at[s, half], rs_recv.at[s, half], right)

        sbuf[...] = pout[0:SQ_SH, :].astype(BF)
        rsx0 = rs_rdma(0, 0)
        rsy0 = rs_rdma(0, 1)
        rsx0.start()
        rsy0.start()

        attn_group(1)
        pout[2 * SQ_SH:4 * SQ_SH, :] = jnp.dot(ctxg[1], wo_bf[...],
                                               preferred_element_type=F32)

        def rs_step(s, prev_x, prev_y, role):
            rr = role * SQ_SH
            prev_x.wait()
            sbuf[:, 0:HD] = (rbuf[s - 1, :, 0:HD].astype(F32)
                             + pout[rr:rr + SQ_SH, 0:HD]).astype(BF)
            nx = rs_rdma(s, 0)
            nx.start()
            prev_y.wait()
            sbuf[:, HD:D_MODEL] = (rbuf[s - 1, :, HD:D_MODEL].astype(F32)
                                   + pout[rr:rr + SQ_SH, HD:D_MODEL]
                                   ).astype(BF)
            ny = rs_rdma(s, 1)
            ny.start()
            return nx, ny

        rsx1, rsy1 = rs_step(1, rsx0, rsy0, 1)
        rsx2, rsy2 = rs_step(2, rsx1, rsy1, 2)
        rsx2.wait()
        out_ref[0, :, 0:HD] = (rbuf[2, :, 0:HD].astype(F32)
                               + pout[3 * SQ_SH:4 * SQ_SH, 0:HD])
        rsy2.wait()
        out_ref[0, :, HD:D_MODEL] = (rbuf[2, :, HD:D_MODEL].astype(F32)
                                     + pout[3 * SQ_SH:4 * SQ_SH, HD:D_MODEL])

    return pl.pallas_call(
        body,
        out_shape=jax.ShapeDtypeStruct((1, SQ_SH, D_MODEL), F32),
        in_specs=[
            pl.BlockSpec(memory_space=pltpu.VMEM),
            pl.BlockSpec(memory_space=pltpu.VMEM),
            pl.BlockSpec(memory_space=pltpu.MemorySpace.HBM),
            pl.BlockSpec(memory_space=pltpu.MemorySpace.HBM),
            pl.BlockSpec(memory_space=pltpu.VMEM),
        ],
        out_specs=pl.BlockSpec(memory_space=pltpu.VMEM),
        scratch_shapes=[
            pltpu.VMEM((N_DEV, SQ_SH, D_MODEL), BF),
            pltpu.VMEM((D_MODEL, H_LOC * DH), BF),
            pltpu.VMEM((H_LOC * DH, D_MODEL), BF),
            pltpu.VMEM((2, H_LOC, 2, SQ_SH, DH), BF),
            pltpu.VMEM((2, H_LOC, 2 * SQ_SH, DH), BF),
            pltpu.VMEM((2, 2 * SQ_SH, H_LOC * DH), BF),
            pltpu.VMEM((SQ, D_MODEL), F32),
            pltpu.VMEM((2, SKV, 1, DH), F32),
            pltpu.VMEM((2, SKV, 1, DH), F32),
            pltpu.VMEM((H_LOC, NPH, KPP, DH), BF),
            pltpu.VMEM((H_LOC, NPH, KPP, DH), BF),
            pltpu.VMEM((H_LOC, KPP, 8), BF),
            pltpu.VMEM((SQ_SH, D_MODEL), BF),
            pltpu.VMEM((N_DEV - 1, SQ_SH, D_MODEL), BF),
            pltpu.SemaphoreType.DMA((2,)),
            pltpu.SemaphoreType.DMA((2,)),
            pltpu.SemaphoreType.DMA((2,)),
            pltpu.SemaphoreType.DMA((2,)),
            pltpu.SemaphoreType.DMA((1,)),
            pltpu.SemaphoreType.DMA((1,)),
            pltpu.SemaphoreType.DMA((N_DEV - 1, 2)),
            pltpu.SemaphoreType.DMA((N_DEV - 1, 2)),
        ],
        compiler_params=pltpu.CompilerParams(
            collective_id=0, vmem_limit_bytes=100 * 1024 * 1024),
    )(x, Wq, K_ext, V_ext, Wo)


# device time: 76075 ns/iter; 1.0525x vs baseline; 1.0525x over previous
import jax
import jax.numpy as jnp
from jax import lax
from jax.experimental import pallas as pl
from jax.experimental.pallas import tpu as pltpu

N_DEV = 4
SQ = 1024
SQ_SH = SQ // N_DEV
H_LOC = 8
DH = 128
SKV = 4096
NPH = 4
KPP = SKV // NPH
D_MODEL = 1024
HD = D_MODEL // 2
SCALE = 0.08838834764831843

BF = jnp.bfloat16
F32 = jnp.float32


def kernel(x, Wq, K_ext, V_ext, Wo):
    def body(x_ref, wq_ref, k_ref, v_ref, wo_ref, out_ref,
             xfull, wq_bf, wo_bf, qh, ctxh, ctxg, pout,
             kst, vst, kre, vre, sbuf, rbuf,
             dma_k, dma_v, agr_send, agr_recv, agl_send, agl_recv,
             rs_send, rs_recv):
        my = lax.axis_index("i")
        left = lax.rem(my + N_DEV - 1, N_DEV)
        right = lax.rem(my + 1, N_DEV)
        hb = my * H_LOC

        barrier = pltpu.get_barrier_semaphore()
        for nbr in (left, right):
            pl.semaphore_signal(barrier, inc=1, device_id=(nbr,),
                                device_id_type=pl.DeviceIdType.MESH)
        pl.semaphore_wait(barrier, 2)

        def kv_dma(h):
            s = h % 2
            dk = pltpu.make_async_copy(
                k_ref.at[0, :, pl.ds(hb + h, 1), :], kst.at[s], dma_k.at[s])
            dv = pltpu.make_async_copy(
                v_ref.at[0, :, pl.ds(hb + h, 1), :], vst.at[s], dma_v.at[s])
            return dk, dv

        def kv_start(h):
            dk, dv = kv_dma(h)
            dk.start()
            dv.start()

        def kv_finish(h):
            dk, dv = kv_dma(h)
            dk.wait()
            dv.wait()
            s = h % 2

            def reorg(a, _):
                for p in range(NPH):
                    src = a * 256 + p * 64
                    kre[h, p, pl.ds(a * 64, 64), :] = \
                        kst[s, pl.ds(src, 64), 0, :].astype(BF)
                    vre[h, p, pl.ds(a * 64, 64), :] = \
                        vst[s, pl.ds(src, 64), 0, :].astype(BF)
                return 0
            lax.fori_loop(0, SKV // 256, reorg, 0)

        def q_chunk(cidx, g, u):
            qc = (jnp.dot(
                xfull[pl.ds(cidx, 1)].reshape(SQ_SH, D_MODEL), wq_bf[...],
                preferred_element_type=F32,
            ) * SCALE).astype(BF)
            for h in range(H_LOC):
                qh[g, h, u] = qc[:, h * DH:(h + 1) * DH]

        def attn_group(g):
            for p in range(NPH):
                qb = qh[g, :, :, p * 64:(p + 1) * 64, :].reshape(
                    H_LOC, 2 * 64, DH)
                s = lax.dot_general(
                    qb, kre[:, p], (((2,), (2,)), ((0,), (0,))),
                    preferred_element_type=F32)
                e = jnp.exp(s)
                denom = jnp.sum(e, axis=2, keepdims=True)
                ctxb = lax.dot_general(
                    e.astype(BF), vre[:, p], (((2,), (1,)), ((0,), (0,))),
                    preferred_element_type=F32)
                val = (ctxb / denom).astype(BF)
                for u in range(2):
                    row = u * SQ_SH + p * 64
                    ctxh[g, :, row:row + 64, :] = val[:, u * 64:(u + 1) * 64, :]
            for h in range(H_LOC):
                ctxg[g, :, h * DH:(h + 1) * DH] = ctxh[g, h]

        def rdma_ring(src, dst, send_sem, recv_sem, target):
            return pltpu.make_async_remote_copy(
                src_ref=src, dst_ref=dst, send_sem=send_sem,
                recv_sem=recv_sem, device_id=(target,),
                device_id_type=pl.DeviceIdType.MESH)

        kv_start(0)
        kv_start(1)
        xfull[pl.ds(my, 1)] = x_ref[...].astype(BF)
        r0 = rdma_ring(xfull.at[pl.ds(my, 1)], xfull.at[pl.ds(my, 1)],
                       agr_send.at[0], agr_recv.at[0], right)
        l0 = rdma_ring(xfull.at[pl.ds(my, 1)], xfull.at[pl.ds(my, 1)],
                       agl_send.at[0], agl_recv.at[0], left)
        r0.start()
        l0.start()
        wq_bf[...] = wq_ref[...].astype(BF)
        wo_bf[...] = wo_ref[...].astype(BF)
        q_chunk(my, 1, 1)
        kv_finish(0)
        kv_start(2)
        r0.wait()
        cm1 = lax.rem(my - 1 + N_DEV, N_DEV)
        q_chunk(cm1, 0, 0)
        r1 = rdma_ring(xfull.at[pl.ds(cm1, 1)], xfull.at[pl.ds(cm1, 1)],
                       agr_send.at[1], agr_recv.at[1], right)
        r1.start()
        kv_finish(1)
        kv_start(3)
        kv_finish(2)
        kv_start(4)
        r1.wait()
        q_chunk(lax.rem(my - 2 + N_DEV, N_DEV), 0, 1)
        kv_finish(3)
        kv_start(5)
        kv_finish(4)
        kv_start(6)
        kv_finish(5)
        kv_start(7)
        kv_finish(6)
        kv_finish(7)

        attn_group(0)
        pout[0:SQ_SH, :] = jnp.dot(ctxg[0, 0:SQ_SH], wo_bf[...],
                                   preferred_element_type=F32)

        def rs_rdma(s, half):
            c0 = half * HD
            return rdma_ring(
                sbuf.at[:, pl.ds(c0, HD)], rbuf.at[s, :, pl.ds(c0, HD)],
                rs_send.at[s, half], rs_recv.at[s, half], right)

        sbuf[...] = pout[0:SQ_SH, :].astype(BF)
        rsx0 = rs_rdma(0, 0)
        rsy0 = rs_rdma(0, 1)
        rsx0.start()
        rsy0.start()

        pout[SQ_SH:2 * SQ_SH, :] = jnp.dot(ctxg[0, SQ_SH:2 * SQ_SH],
                                           wo_bf[...],
                                           preferred_element_type=F32)
        l0.wait()
        q_chunk(lax.rem(my + 1, N_DEV), 1, 0)
        attn_group(1)
        pout[2 * SQ_SH:3 * SQ_SH, :] = jnp.dot(ctxg[1, 0:SQ_SH], wo_bf[...],
                                               preferred_element_type=F32)

        def rs_step(s, prev_x, prev_y, role):
            rr = role * SQ_SH
            prev_x.wait()
            sbuf[:, 0:HD] = (rbuf[s - 1, :, 0:HD].astype(F32)
                             + pout[rr:rr + SQ_SH, 0:HD]).astype(BF)
            nx = rs_rdma(s, 0)
            nx.start()
            prev_y.wait()
            sbuf[:, HD:D_MODEL] = (rbuf[s - 1, :, HD:D_MODEL].astype(F32)
                                   + pout[rr:rr + SQ_SH, HD:D_MODEL]
                                   ).astype(BF)
            ny = rs_rdma(s, 1)
            ny.start()
            return nx, ny

        rsx1, rsy1 = rs_step(1, rsx0, rsy0, 1)
        pout[3 * SQ_SH:4 * SQ_SH, :] = jnp.dot(ctxg[1, SQ_SH:2 * SQ_SH],
                                               wo_bf[...],
                                               preferred_element_type=F32)
        rsx2, rsy2 = rs_step(2, rsx1, rsy1, 2)
        rsx2.wait()
        out_ref[0, :, 0:HD] = (rbuf[2, :, 0:HD].astype(F32)
                               + pout[3 * SQ_SH:4 * SQ_SH, 0:HD])
        rsy2.wait()
        out_ref[0, :, HD:D_MODEL] = (rbuf[2, :, HD:D_MODEL].astype(F32)
                                     + pout[3 * SQ_SH:4 * SQ_SH, HD:D_MODEL])

    return pl.pallas_call(
        body,
        out_shape=jax.ShapeDtypeStruct((1, SQ_SH, D_MODEL), F32),
        in_specs=[
            pl.BlockSpec(memory_space=pltpu.VMEM),
            pl.BlockSpec(memory_space=pltpu.VMEM),
            pl.BlockSpec(memory_space=pltpu.MemorySpace.HBM),
            pl.BlockSpec(memory_space=pltpu.MemorySpace.HBM),
            pl.BlockSpec(memory_space=pltpu.VMEM),
        ],
        out_specs=pl.BlockSpec(memory_space=pltpu.VMEM),
        scratch_shapes=[
            pltpu.VMEM((N_DEV, SQ_SH, D_MODEL), BF),
            pltpu.VMEM((D_MODEL, H_LOC * DH), BF),
            pltpu.VMEM((H_LOC * DH, D_MODEL), BF),
            pltpu.VMEM((2, H_LOC, 2, SQ_SH, DH), BF),
            pltpu.VMEM((2, H_LOC, 2 * SQ_SH, DH), BF),
            pltpu.VMEM((2, 2 * SQ_SH, H_LOC * DH), BF),
            pltpu.VMEM((SQ, D_MODEL), F32),
            pltpu.VMEM((2, SKV, 1, DH), F32),
            pltpu.VMEM((2, SKV, 1, DH), F32),
            pltpu.VMEM((H_LOC, NPH, KPP, DH), BF),
            pltpu.VMEM((H_LOC, NPH, KPP, DH), BF),
            pltpu.VMEM((SQ_SH, D_MODEL), BF),
            pltpu.VMEM((N_DEV - 1, SQ_SH, D_MODEL), BF),
            pltpu.SemaphoreType.DMA((2,)),
            pltpu.SemaphoreType.DMA((2,)),
            pltpu.SemaphoreType.DMA((2,)),
            pltpu.SemaphoreType.DMA((2,)),
            pltpu.SemaphoreType.DMA((1,)),
            pltpu.SemaphoreType.DMA((1,)),
            pltpu.SemaphoreType.DMA((N_DEV - 1, 2)),
            pltpu.SemaphoreType.DMA((N_DEV - 1, 2)),
        ],
        compiler_params=pltpu.CompilerParams(
            collective_id=0, vmem_limit_bytes=100 * 1024 * 1024),
    )(x, Wq, K_ext, V_ext, Wo)


# device time: 75890 ns/iter; 1.0550x vs baseline; 1.0024x over previous
import jax
import jax.numpy as jnp
from jax import lax
from jax.experimental import pallas as pl
from jax.experimental.pallas import tpu as pltpu

N_DEV = 4
SQ = 1024
SQ_SH = SQ // N_DEV
H_LOC = 8
DH = 128
SKV = 4096
NPH = 4
KPP = SKV // NPH
D_MODEL = 1024
HD = D_MODEL // 2
SCALE = 0.08838834764831843

BF = jnp.bfloat16
F32 = jnp.float32


def kernel(x, Wq, K_ext, V_ext, Wo):
    def body(x_ref, wq_ref, k_ref, v_ref, wo_ref, out_ref,
             xfull, wq_bf, wo_bf, qh, ctxh, ctxg, pout,
             kst, vst, kre, vre, sbuf, rbuf,
             dma_k, dma_v, agr_send, agr_recv, agl_send, agl_recv,
             rs_send, rs_recv):
        my = lax.axis_index("i")
        left = lax.rem(my + N_DEV - 1, N_DEV)
        right = lax.rem(my + 1, N_DEV)
        hb = my * H_LOC

        barrier = pltpu.get_barrier_semaphore()
        for nbr in (left, right):
            pl.semaphore_signal(barrier, inc=1, device_id=(nbr,),
                                device_id_type=pl.DeviceIdType.MESH)
        pl.semaphore_wait(barrier, 2)

        def kv_dma(h):
            s = h % 2
            dk = pltpu.make_async_copy(
                k_ref.at[0, :, pl.ds(hb + h, 1), :], kst.at[s], dma_k.at[s])
            dv = pltpu.make_async_copy(
                v_ref.at[0, :, pl.ds(hb + h, 1), :], vst.at[s], dma_v.at[s])
            return dk, dv

        def kv_start(h):
            dk, dv = kv_dma(h)
            dk.start()
            dv.start()

        def kv_finish(h):
            dk, dv = kv_dma(h)
            dk.wait()
            dv.wait()
            s = h % 2

            def reorg(a, _):
                kval = kst[s, pl.ds(a * 256, 256), 0, :].astype(BF)
                vval = vst[s, pl.ds(a * 256, 256), 0, :].astype(BF)
                for p in range(NPH):
                    kre[h, p, pl.ds(a * 64, 64), :] = kval[p * 64:(p + 1) * 64]
                    vre[h, p, pl.ds(a * 64, 64), :] = vval[p * 64:(p + 1) * 64]
                return 0
            lax.fori_loop(0, SKV // 256, reorg, 0)

        def q_chunk(cidx, g, u):
            qc = (jnp.dot(
                xfull[pl.ds(cidx, 1)].reshape(SQ_SH, D_MODEL), wq_bf[...],
                preferred_element_type=F32,
            ) * SCALE).astype(BF)
            for h in range(H_LOC):
                qh[g, h, u] = qc[:, h * DH:(h + 1) * DH]

        def attn_group(g):
            for p in range(NPH):
                qb = qh[g, :, :, p * 64:(p + 1) * 64, :].reshape(
                    H_LOC, 2 * 64, DH)
                s = lax.dot_general(
                    qb, kre[:, p], (((2,), (2,)), ((0,), (0,))),
                    preferred_element_type=F32)
                e = jnp.exp(s)
                denom = jnp.sum(e, axis=2, keepdims=True)
                ctxb = lax.dot_general(
                    e.astype(BF), vre[:, p], (((2,), (1,)), ((0,), (0,))),
                    preferred_element_type=F32)
                val = (ctxb / denom).astype(BF)
                for u in range(2):
                    row = u * SQ_SH + p * 64
                    ctxh[g, :, row:row + 64, :] = val[:, u * 64:(u + 1) * 64, :]
            for h in range(H_LOC):
                ctxg[g, :, h * DH:(h + 1) * DH] = ctxh[g, h]

        def rdma_ring(src, dst, send_sem, recv_sem, target):
            return pltpu.make_async_remote_copy(
                src_ref=src, dst_ref=dst, send_sem=send_sem,
                recv_sem=recv_sem, device_id=(target,),
                device_id_type=pl.DeviceIdType.MESH)

        kv_start(0)
        kv_start(1)
        xfull[pl.ds(my, 1)] = x_ref[...].astype(BF)
        r0 = rdma_ring(xfull.at[pl.ds(my, 1)], xfull.at[pl.ds(my, 1)],
                       agr_send.at[0], agr_recv.at[0], right)
        l0 = rdma_ring(xfull.at[pl.ds(my, 1)], xfull.at[pl.ds(my, 1)],
                       agl_send.at[0], agl_recv.at[0], left)
        r0.start()
        l0.start()
        wq_bf[...] = wq_ref[...].astype(BF)
        wo_bf[...] = wo_ref[...].astype(BF)
        q_chunk(my, 1, 1)
        kv_finish(0)
        kv_start(2)
        r0.wait()
        cm1 = lax.rem(my - 1 + N_DEV, N_DEV)
        q_chunk(cm1, 0, 0)
        r1 = rdma_ring(xfull.at[pl.ds(cm1, 1)], xfull.at[pl.ds(cm1, 1)],
                       agr_send.at[1], agr_recv.at[1], right)
        r1.start()
        kv_finish(1)
        kv_start(3)
        kv_finish(2)
        kv_start(4)
        r1.wait()
        q_chunk(lax.rem(my - 2 + N_DEV, N_DEV), 0, 1)
        kv_finish(3)
        kv_start(5)
        kv_finish(4)
        kv_start(6)
        kv_finish(5)
        kv_start(7)
        kv_finish(6)
        kv_finish(7)

        attn_group(0)
        pout[0:SQ_SH, :] = jnp.dot(ctxg[0, 0:SQ_SH], wo_bf[...],
                                   preferred_element_type=F32)

        def rs_rdma(s, half):
            c0 = half * HD
            return rdma_ring(
                sbuf.at[:, pl.ds(c0, HD)], rbuf.at[s, :, pl.ds(c0, HD)],
                rs_send.at[s, half], rs_recv.at[s, half], right)

        sbuf[...] = pout[0:SQ_SH, :].astype(BF)
        rsx0 = rs_rdma(0, 0)
        rsy0 = rs_rdma(0, 1)
        rsx0.start()
        rsy0.start()

        pout[SQ_SH:2 * SQ_SH, :] = jnp.dot(ctxg[0, SQ_SH:2 * SQ_SH],
                                           wo_bf[...],
                                           preferred_element_type=F32)
        l0.wait()
        q_chunk(lax.rem(my + 1, N_DEV), 1, 0)
        attn_group(1)
        pout[2 * SQ_SH:3 * SQ_SH, :] = jnp.dot(ctxg[1, 0:SQ_SH], wo_bf[...],
                                               preferred_element_type=F32)

        def rs_step(s, prev_x, prev_y, role):
            rr = role * SQ_SH
            prev_x.wait()
            sbuf[:, 0:HD] = (rbuf[s - 1, :, 0:HD].astype(F32)
                             + pout[rr:rr + SQ_SH, 0:HD]).astype(BF)
            nx = rs_rdma(s, 0)
            nx.start()
            prev_y.wait()
            sbuf[:, HD:D_MODEL] = (rbuf[s - 1, :, HD:D_MODEL].astype(F32)
                                   + pout[rr:rr + SQ_SH, HD:D_MODEL]
                                   ).astype(BF)
            ny = rs_rdma(s, 1)
            ny.start()
            return nx, ny

        rsx1, rsy1 = rs_step(1, rsx0, rsy0, 1)
        pout[3 * SQ_SH:4 * SQ_SH, :] = jnp.dot(ctxg[1, SQ_SH:2 * SQ_SH],
                                               wo_bf[...],
                                               preferred_element_type=F32)
        rsx2, rsy2 = rs_step(2, rsx1, rsy1, 2)
        rsx2.wait()
        out_ref[0, :, 0:HD] = (rbuf[2, :, 0:HD].astype(F32)
                               + pout[3 * SQ_SH:4 * SQ_SH, 0:HD])
        rsy2.wait()
        out_ref[0, :, HD:D_MODEL] = (rbuf[2, :, HD:D_MODEL].astype(F32)
                                     + pout[3 * SQ_SH:4 * SQ_SH, HD:D_MODEL])

    return pl.pallas_call(
        body,
        out_shape=jax.ShapeDtypeStruct((1, SQ_SH, D_MODEL), F32),
        in_specs=[
            pl.BlockSpec(memory_space=pltpu.VMEM),
            pl.BlockSpec(memory_space=pltpu.VMEM),
            pl.BlockSpec(memory_space=pltpu.MemorySpace.HBM),
            pl.BlockSpec(memory_space=pltpu.MemorySpace.HBM),
            pl.BlockSpec(memory_space=pltpu.VMEM),
        ],
        out_specs=pl.BlockSpec(memory_space=pltpu.VMEM),
        scratch_shapes=[
            pltpu.VMEM((N_DEV, SQ_SH, D_MODEL), BF),
            pltpu.VMEM((D_MODEL, H_LOC * DH), BF),
            pltpu.VMEM((H_LOC * DH, D_MODEL), BF),
            pltpu.VMEM((2, H_LOC, 2, SQ_SH, DH), BF),
            pltpu.VMEM((2, H_LOC, 2 * SQ_SH, DH), BF),
            pltpu.VMEM((2, 2 * SQ_SH, H_LOC * DH), BF),
            pltpu.VMEM((SQ, D_MODEL), F32),
            pltpu.VMEM((2, SKV, 1, DH), F32),
            pltpu.VMEM((2, SKV, 1, DH), F32),
            pltpu.VMEM((H_LOC, NPH, KPP, DH), BF),
            pltpu.VMEM((H_LOC, NPH, KPP, DH), BF),
            pltpu.VMEM((SQ_SH, D_MODEL), BF),
            pltpu.VMEM((N_DEV - 1, SQ_SH, D_MODEL), BF),
            pltpu.SemaphoreType.DMA((2,)),
            pltpu.SemaphoreType.DMA((2,)),
            pltpu.SemaphoreType.DMA((2,)),
            pltpu.SemaphoreType.DMA((2,)),
            pltpu.SemaphoreType.DMA((1,)),
            pltpu.SemaphoreType.DMA((1,)),
            pltpu.SemaphoreType.DMA((N_DEV - 1, 2)),
            pltpu.SemaphoreType.DMA((N_DEV - 1, 2)),
        ],
        compiler_params=pltpu.CompilerParams(
            collective_id=0, vmem_limit_bytes=100 * 1024 * 1024),
    )(x, Wq, K_ext, V_ext, Wo)


# device time: 69786 ns/iter; 1.1473x vs baseline; 1.0875x over previous
import jax
import jax.numpy as jnp
from jax import lax
from jax.experimental import pallas as pl
from jax.experimental.pallas import tpu as pltpu

N_DEV = 4
SQ = 1024
SQ_SH = SQ // N_DEV
H_LOC = 8
DH = 128
SKV = 4096
NPH = 4
KPP = SKV // NPH
D_MODEL = 1024
HD = D_MODEL // 2
SCALE = 0.08838834764831843

BF = jnp.bfloat16
F32 = jnp.float32


def kernel(x, Wq, K_ext, V_ext, Wo):
    def body(x_ref, wq_ref, k_ref, v_ref, wo_ref, out_ref,
             xfull, wq_bf, wo_bf, qh, ctxh, ctxg, pout,
             kst, vst, kh2, vh2, kre, vre, sbuf, rbuf,
             dma_k, dma_v, tk_sem, tv_sem,
             agr_send, agr_recv, agl_send, agl_recv,
             rs_send, rs_recv):
        my = lax.axis_index("i")
        left = lax.rem(my + N_DEV - 1, N_DEV)
        right = lax.rem(my + 1, N_DEV)
        hb = my * H_LOC

        barrier = pltpu.get_barrier_semaphore()
        for nbr in (left, right):
            pl.semaphore_signal(barrier, inc=1, device_id=(nbr,),
                                device_id_type=pl.DeviceIdType.MESH)
        pl.semaphore_wait(barrier, 2)

        KVC = 512
        def kv_dma(c):
            s = c % 2
            dk = pltpu.make_async_copy(
                k_ref.at[0, pl.ds(c * KVC, KVC), pl.ds(hb, H_LOC), :],
                kst.at[s], dma_k.at[s])
            dv = pltpu.make_async_copy(
                v_ref.at[0, pl.ds(c * KVC, KVC), pl.ds(hb, H_LOC), :],
                vst.at[s], dma_v.at[s])
            return dk, dv

        def kv_start(c):
            dk, dv = kv_dma(c)
            dk.start()
            dv.start()

        def kv_finish(c):
            dk, dv = kv_dma(c)
            dk.wait()
            dv.wait()
            s = c % 2
            tks = []
            for h in range(H_LOC):
                tk = pltpu.make_async_copy(
                    kst.at[s, :, h, :], kh2.at[h], tk_sem.at[h])
                tv = pltpu.make_async_copy(
                    vst.at[s, :, h, :], vh2.at[h], tv_sem.at[h])
                tk.start()
                tv.start()
                tks.append((tk, tv))
            for h in range(H_LOC):
                tk, tv = tks[h]
                tk.wait()
                tv.wait()
                for half in range(2):
                    kval = kh2[h, half * 256:(half + 1) * 256, :].astype(BF)
                    vval = vh2[h, half * 256:(half + 1) * 256, :].astype(BF)
                    for q in range(NPH):
                        j = half * 4 + q
                        p = j % 4
                        dst = (2 * c + j // 4) * 64
                        kre[h, p, dst:dst + 64, :] = kval[q * 64:(q + 1) * 64]
                        vre[h, p, dst:dst + 64, :] = vval[q * 64:(q + 1) * 64]

        def q_chunk(cidx, g, u):
            qc = (jnp.dot(
                xfull[pl.ds(cidx, 1)].reshape(SQ_SH, D_MODEL), wq_bf[...],
                preferred_element_type=F32,
            ) * SCALE).astype(BF)
            for h in range(H_LOC):
                qh[g, h, u] = qc[:, h * DH:(h + 1) * DH]

        def attn_group(g):
            for p in range(NPH):
                qb = qh[g, :, :, p * 64:(p + 1) * 64, :].reshape(
                    H_LOC, 2 * 64, DH)
                s = lax.dot_general(
                    qb, kre[:, p], (((2,), (2,)), ((0,), (0,))),
                    preferred_element_type=F32)
                e = jnp.exp(s)
                denom = jnp.sum(e, axis=2, keepdims=True)
                ctxb = lax.dot_general(
                    e.astype(BF), vre[:, p], (((2,), (1,)), ((0,), (0,))),
                    preferred_element_type=F32)
                val = (ctxb / denom).astype(BF)
                for u in range(2):
                    row = u * SQ_SH + p * 64
                    ctxh[g, :, row:row + 64, :] = val[:, u * 64:(u + 1) * 64, :]
            for h in range(H_LOC):
                ctxg[g, :, h * DH:(h + 1) * DH] = ctxh[g, h]

        def rdma_ring(src, dst, send_sem, recv_sem, target):
            return pltpu.make_async_remote_copy(
                src_ref=src, dst_ref=dst, send_sem=send_sem,
                recv_sem=recv_sem, device_id=(target,),
                device_id_type=pl.DeviceIdType.MESH)

        kv_start(0)
        kv_start(1)
        xfull[pl.ds(my, 1)] = x_ref[...].astype(BF)
        r0 = rdma_ring(xfull.at[pl.ds(my, 1)], xfull.at[pl.ds(my, 1)],
                       agr_send.at[0], agr_recv.at[0], right)
        l0 = rdma_ring(xfull.at[pl.ds(my, 1)], xfull.at[pl.ds(my, 1)],
                       agl_send.at[0], agl_recv.at[0], left)
        r0.start()
        l0.start()
        wq_bf[...] = wq_ref[...].astype(BF)
        wo_bf[...] = wo_ref[...].astype(BF)
        q_chunk(my, 1, 1)
        kv_finish(0)
        kv_start(2)
        r0.wait()
        cm1 = lax.rem(my - 1 + N_DEV, N_DEV)
        q_chunk(cm1, 0, 0)
        r1 = rdma_ring(xfull.at[pl.ds(cm1, 1)], xfull.at[pl.ds(cm1, 1)],
                       agr_send.at[1], agr_recv.at[1], right)
        r1.start()
        kv_finish(1)
        kv_start(3)
        kv_finish(2)
        kv_start(4)
        r1.wait()
        q_chunk(lax.rem(my - 2 + N_DEV, N_DEV), 0, 1)
        kv_finish(3)
        kv_start(5)
        kv_finish(4)
        kv_start(6)
        kv_finish(5)
        kv_start(7)
        kv_finish(6)
        kv_finish(7)

        attn_group(0)
        pout[0:SQ_SH, :] = jnp.dot(ctxg[0, 0:SQ_SH], wo_bf[...],
                                   preferred_element_type=F32)

        def rs_rdma(s, half):
            c0 = half * HD
            return rdma_ring(
                sbuf.at[:, pl.ds(c0, HD)], rbuf.at[s, :, pl.ds(c0, HD)],
                rs_send.at[s, half], rs_recv.at[s, half], right)

        sbuf[...] = pout[0:SQ_SH, :].astype(BF)
        rsx0 = rs_rdma(0, 0)
        rsy0 = rs_rdma(0, 1)
        rsx0.start()
        rsy0.start()

        pout[SQ_SH:2 * SQ_SH, :] = jnp.dot(ctxg[0, SQ_SH:2 * SQ_SH],
                                           wo_bf[...],
                                           preferred_element_type=F32)
        l0.wait()
        q_chunk(lax.rem(my + 1, N_DEV), 1, 0)
        attn_group(1)
        pout[2 * SQ_SH:3 * SQ_SH, :] = jnp.dot(ctxg[1, 0:SQ_SH], wo_bf[...],
                                               preferred_element_type=F32)

        def rs_step(s, prev_x, prev_y, role):
            rr = role * SQ_SH
            prev_x.wait()
            sbuf[:, 0:HD] = (rbuf[s - 1, :, 0:HD].astype(F32)
                             + pout[rr:rr + SQ_SH, 0:HD]).astype(BF)
            nx = rs_rdma(s, 0)
            nx.start()
            prev_y.wait()
            sbuf[:, HD:D_MODEL] = (rbuf[s - 1, :, HD:D_MODEL].astype(F32)
                                   + pout[rr:rr + SQ_SH, HD:D_MODEL]
                                   ).astype(BF)
            ny = rs_rdma(s, 1)
            ny.start()
            return nx, ny

        rsx1, rsy1 = rs_step(1, rsx0, rsy0, 1)
        pout[3 * SQ_SH:4 * SQ_SH, :] = jnp.dot(ctxg[1, SQ_SH:2 * SQ_SH],
                                               wo_bf[...],
                                               preferred_element_type=F32)
        rsx2, rsy2 = rs_step(2, rsx1, rsy1, 2)
        rsx2.wait()
        out_ref[0, :, 0:HD] = (rbuf[2, :, 0:HD].astype(F32)
                               + pout[3 * SQ_SH:4 * SQ_SH, 0:HD])
        rsy2.wait()
        out_ref[0, :, HD:D_MODEL] = (rbuf[2, :, HD:D_MODEL].astype(F32)
                                     + pout[3 * SQ_SH:4 * SQ_SH, HD:D_MODEL])

    return pl.pallas_call(
        body,
        out_shape=jax.ShapeDtypeStruct((1, SQ_SH, D_MODEL), F32),
        in_specs=[
            pl.BlockSpec(memory_space=pltpu.VMEM),
            pl.BlockSpec(memory_space=pltpu.VMEM),
            pl.BlockSpec(memory_space=pltpu.MemorySpace.HBM),
            pl.BlockSpec(memory_space=pltpu.MemorySpace.HBM),
            pl.BlockSpec(memory_space=pltpu.VMEM),
        ],
        out_specs=pl.BlockSpec(memory_space=pltpu.VMEM),
        scratch_shapes=[
            pltpu.VMEM((N_DEV, SQ_SH, D_MODEL), BF),
            pltpu.VMEM((D_MODEL, H_LOC * DH), BF),
            pltpu.VMEM((H_LOC * DH, D_MODEL), BF),
            pltpu.VMEM((2, H_LOC, 2, SQ_SH, DH), BF),
            pltpu.VMEM((2, H_LOC, 2 * SQ_SH, DH), BF),
            pltpu.VMEM((2, 2 * SQ_SH, H_LOC * DH), BF),
            pltpu.VMEM((SQ, D_MODEL), F32),
            pltpu.VMEM((2, 512, H_LOC, DH), F32),
            pltpu.VMEM((2, 512, H_LOC, DH), F32),
            pltpu.VMEM((H_LOC, 512, DH), F32),
            pltpu.VMEM((H_LOC, 512, DH), F32),
            pltpu.VMEM((H_LOC, NPH, KPP, DH), BF),
            pltpu.VMEM((H_LOC, NPH, KPP, DH), BF),
            pltpu.VMEM((SQ_SH, D_MODEL), BF),
            pltpu.VMEM((N_DEV - 1, SQ_SH, D_MODEL), BF),
            pltpu.SemaphoreType.DMA((2,)),
            pltpu.SemaphoreType.DMA((2,)),
            pltpu.SemaphoreType.DMA((H_LOC,)),
            pltpu.SemaphoreType.DMA((H_LOC,)),
            pltpu.SemaphoreType.DMA((2,)),
            pltpu.SemaphoreType.DMA((2,)),
            pltpu.SemaphoreType.DMA((1,)),
            pltpu.SemaphoreType.DMA((1,)),
            pltpu.SemaphoreType.DMA((N_DEV - 1, 2)),
            pltpu.SemaphoreType.DMA((N_DEV - 1, 2)),
        ],
        compiler_params=pltpu.CompilerParams(
            collective_id=0, vmem_limit_bytes=100 * 1024 * 1024),
    )(x, Wq, K_ext, V_ext, Wo)
